# baseline (speedup 1.0000x reference)
import jax
import jax.numpy as jnp
from jax import lax
from jax.experimental import pallas as pl
from jax.experimental.pallas import tpu as pltpu
from jax.experimental.pallas import tpu_sc as plsc

_NSRC = 32
_NY = 1024
_NX = 2048
_L = 16

_SC_PARAMS = pltpu.CompilerParams(
    use_tc_tiling_on_sc=True, needs_layout_passes=False
)
_SC_MESH = plsc.VectorSubcoreMesh(core_axis_name="c", subcore_axis_name="s")


def _extract_coords(pk, wid):
    lanes = lax.iota(jnp.int32, _L)
    zero = jnp.zeros((_L,), jnp.int32)
    yi = jnp.int32(0)
    xi = jnp.int32(0)
    for c in range(_NSRC // _L):
        m = (lanes + c * _L) == wid
        yi = yi + jnp.sum(jnp.where(m, pk[pl.ds(c * _L, _L)], zero))
        xi = xi + jnp.sum(jnp.where(m, pk[pl.ds(_NSRC + c * _L, _L)], zero))
    return yi, xi


def _wid():
    return lax.axis_index("s") * 2 + lax.axis_index("c")


def _gather_body(y2_hbm, pk_hbm, tiles_hbm, pk, tile, sem):
    wid = _wid()
    pltpu.sync_copy(pk_hbm, pk)
    yi, xi = _extract_coords(pk, wid)
    upd = plsc.bitcast(pk[pl.ds(2 * _NSRC, _L)], jnp.float32)

    row0 = wid * _NY + (yi >> 3) * 8
    col0 = (xi >> 7) * 128
    ry = yi & 7
    c0 = (xi & 127) & ~15
    lane = xi & 15

    pltpu.async_copy(y2_hbm.at[pl.ds(row0, 8), pl.ds(col0, 128)], tile, sem).wait()
    sel = lax.iota(jnp.int32, _L) == lane
    delta = jnp.where(sel, upd, jnp.float32(0.0))
    tile[ry, pl.ds(c0, _L)] = tile[ry, pl.ds(c0, _L)] + delta
    pltpu.async_copy(tile, tiles_hbm.at[wid], sem).wait()


_gather_add = pl.kernel(
    _gather_body,
    out_type=jax.ShapeDtypeStruct((_NSRC, 8, 128), jnp.float32),
    mesh=_SC_MESH,
    scratch_types=[
        pltpu.VMEM((2 * _NSRC + _L,), jnp.int32),
        pltpu.VMEM((8, 128), jnp.float32),
        pltpu.SemaphoreType.DMA,
    ],
    compiler_params=_SC_PARAMS,
)


def _scatter_body(yref, tiles_hbm, pk_hbm, pk, tile, sem):
    wid = _wid()
    pltpu.async_copy(tiles_hbm.at[wid], tile, sem)
    cp = pltpu.async_copy(pk_hbm, pk, sem)
    cp.wait()
    pltpu.make_async_copy(tiles_hbm.at[wid], tile, sem).wait()
    yi, xi = _extract_coords(pk, wid)
    row0 = wid * _NY + (yi >> 3) * 8
    col0 = (xi >> 7) * 128
    pltpu.async_copy(tile, yref.at[pl.ds(row0, 8), pl.ds(col0, 128)], sem).wait()


_scatter_tiles = pl.kernel(
    _scatter_body,
    out_type=(),
    mesh=_SC_MESH,
    scratch_types=[
        pltpu.VMEM((2 * _NSRC + _L,), jnp.int32),
        pltpu.VMEM((8, 128), jnp.float32),
        pltpu.SemaphoreType.DMA,
    ],
    compiler_params=_SC_PARAMS,
)


def kernel(Y, X, y, x, dt=1.0):
    upd = jnp.asarray(dt, jnp.float32) * X.astype(jnp.float32).reshape(())
    upd16 = jnp.broadcast_to(upd, (_L,))
    packed = jnp.concatenate(
        [
            y.astype(jnp.int32),
            x.astype(jnp.int32),
            lax.bitcast_convert_type(upd16, jnp.int32),
        ]
    )
    Y2 = Y.reshape(_NSRC * _NY, _NX)
    tiles = _gather_add(Y2, packed)
    yref = jax.new_ref(Y2)
    _scatter_tiles(yref, tiles, packed)
    return jax.freeze(yref).reshape(_NSRC, _NY, _NX)


# speedup vs baseline: 1.0243x; 1.0243x over previous
import jax
import jax.numpy as jnp
from jax import lax
from jax.experimental import pallas as pl
from jax.experimental.pallas import tpu as pltpu
from jax.experimental.pallas import tpu_sc as plsc

_NSRC = 32
_NY = 1024
_NX = 2048
_L = 16


def _sc_body(yref, pk_hbm, pk, tile, sem):
    cid = lax.axis_index("c")
    sid = lax.axis_index("s")
    wid = sid * 2 + cid

    pltpu.sync_copy(pk_hbm, pk)

    lanes = lax.iota(jnp.int32, _L)
    zero = jnp.zeros((_L,), jnp.int32)
    yi = jnp.int32(0)
    xi = jnp.int32(0)
    for c in range(_NSRC // _L):
        m = (lanes + c * _L) == wid
        yi = yi + jnp.sum(jnp.where(m, pk[pl.ds(c * _L, _L)], zero))
        xi = xi + jnp.sum(jnp.where(m, pk[pl.ds(_NSRC + c * _L, _L)], zero))
    upd = plsc.bitcast(pk[pl.ds(2 * _NSRC, _L)], jnp.float32)

    row0 = wid * _NY + (yi >> 3) * 8
    col0 = (xi >> 7) * 128
    ry = yi & 7
    c0 = (xi & 127) & ~15
    lane = xi & 15

    pltpu.async_copy(yref.at[pl.ds(row0, 8), pl.ds(col0, 128)], tile, sem).wait()
    sel = lanes == lane
    delta = jnp.where(sel, upd, jnp.float32(0.0))
    tile[ry, pl.ds(c0, _L)] = tile[ry, pl.ds(c0, _L)] + delta
    pltpu.async_copy(tile, yref.at[pl.ds(row0, 8), pl.ds(col0, 128)], sem).wait()


_scatter_add = pl.kernel(
    _sc_body,
    out_type=(),
    mesh=plsc.VectorSubcoreMesh(core_axis_name="c", subcore_axis_name="s"),
    scratch_types=[
        pltpu.VMEM((2 * _NSRC + _L,), jnp.int32),
        pltpu.VMEM((8, 128), jnp.float32),
        pltpu.SemaphoreType.DMA,
    ],
    compiler_params=pltpu.CompilerParams(
        use_tc_tiling_on_sc=True, needs_layout_passes=False
    ),
)


def kernel(Y, X, y, x, dt=1.0):
    upd = jnp.asarray(dt, jnp.float32) * X.astype(jnp.float32).reshape(())
    upd16 = jnp.broadcast_to(upd, (_L,))
    packed = jnp.concatenate(
        [
            y.astype(jnp.int32),
            x.astype(jnp.int32),
            lax.bitcast_convert_type(upd16, jnp.int32),
        ]
    )
    yref = jax.new_ref(Y.reshape(_NSRC * _NY, _NX))
    _scatter_add(yref, packed)
    return jax.freeze(yref).reshape(_NSRC, _NY, _NX)


# speedup vs baseline: 1.0256x; 1.0013x over previous
import jax
import jax.numpy as jnp
from jax import lax
from jax.experimental import pallas as pl
from jax.experimental.pallas import tpu as pltpu
from jax.experimental.pallas import tpu_sc as plsc

_NSRC = 32
_NY = 1024
_NX = 2048
_L = 16


def _sc_body(yref, pk_hbm, pk, tile, sem):
    cid = lax.axis_index("c")
    sid = lax.axis_index("s")
    wid = sid * 2 + cid

    pltpu.sync_copy(pk_hbm, pk)

    lanes = lax.iota(jnp.int32, _L)
    zero = jnp.zeros((_L,), jnp.int32)
    yi = jnp.int32(0)
    xi = jnp.int32(0)
    for c in range(_NSRC // _L):
        m = (lanes + c * _L) == wid
        yi = yi + jnp.sum(jnp.where(m, pk[pl.ds(c * _L, _L)], zero))
        xi = xi + jnp.sum(jnp.where(m, pk[pl.ds(_NSRC + c * _L, _L)], zero))
    upd = plsc.bitcast(pk[pl.ds(2 * _NSRC, _L)], jnp.float32)

    row0 = wid * _NY + (yi >> 3) * 8
    col0 = (xi >> 7) * 128
    ry = yi & 7
    c0 = (xi & 127) & ~15
    lane = xi & 15

    pltpu.async_copy(yref.at[pl.ds(row0, 8), pl.ds(col0, 128)], tile, sem).wait()
    sel = lanes == lane
    delta = jnp.where(sel, upd, jnp.float32(0.0))
    tile[ry, pl.ds(c0, _L)] = tile[ry, pl.ds(c0, _L)] + delta
    pltpu.async_copy(tile, yref.at[pl.ds(row0, 8), pl.ds(col0, 128)], sem).wait()


_scatter_add = pl.kernel(
    _sc_body,
    out_type=(),
    mesh=plsc.VectorSubcoreMesh(core_axis_name="c", subcore_axis_name="s"),
    scratch_types=[
        pltpu.VMEM((2 * _NSRC + _L,), jnp.int32),
        pltpu.VMEM((8, 128), jnp.float32),
        pltpu.SemaphoreType.DMA,
    ],
    compiler_params=pltpu.CompilerParams(
        use_tc_tiling_on_sc=True,
        needs_layout_passes=False,
        skip_device_barrier=True,
        disable_semaphore_checks=True,
    ),
)


def kernel(Y, X, y, x, dt=1.0):
    upd = jnp.asarray(dt, jnp.float32) * X.astype(jnp.float32).reshape(())
    upd16 = jnp.broadcast_to(upd, (_L,))
    packed = jnp.concatenate(
        [
            y.astype(jnp.int32),
            x.astype(jnp.int32),
            lax.bitcast_convert_type(upd16, jnp.int32),
        ]
    )
    yref = jax.new_ref(Y.reshape(_NSRC * _NY, _NX))
    _scatter_add(yref, packed)
    return jax.freeze(yref).reshape(_NSRC, _NY, _NX)


# speedup vs baseline: 1.0351x; 1.0092x over previous
import jax
import jax.numpy as jnp
from jax import lax
from jax.experimental import pallas as pl
from jax.experimental.pallas import tpu as pltpu
from jax.experimental.pallas import tpu_sc as plsc

_NSRC = 32
_NY = 1024
_NX = 2048
_L = 16


def _sc_body(yref, y_hbm, x_hbm, xs_hbm, dt_hbm, yv, xv, xs, dtv, tile, sem):
    cid = lax.axis_index("c")
    sid = lax.axis_index("s")
    wid = sid * 2 + cid

    pltpu.async_copy(y_hbm, yv, sem)
    pltpu.async_copy(x_hbm, xv, sem)
    pltpu.async_copy(xs_hbm, xs, sem)
    pltpu.async_copy(dt_hbm, dtv, sem)
    pltpu.make_async_copy(y_hbm, yv, sem).wait()
    pltpu.make_async_copy(x_hbm, xv, sem).wait()
    pltpu.make_async_copy(xs_hbm, xs, sem).wait()
    pltpu.make_async_copy(dt_hbm, dtv, sem).wait()

    lanes = lax.iota(jnp.int32, _L)
    zero = jnp.zeros((_L,), jnp.int32)
    yi = jnp.int32(0)
    xi = jnp.int32(0)
    for c in range(_NSRC // _L):
        m = (lanes + c * _L) == wid
        yi = yi + jnp.sum(jnp.where(m, yv[pl.ds(c * _L, _L)], zero))
        xi = xi + jnp.sum(jnp.where(m, xv[pl.ds(c * _L, _L)], zero))
    zidx = jnp.zeros((_L,), jnp.int32)
    upd = plsc.load_gather(xs, [zidx]) * plsc.load_gather(dtv, [zidx])

    row0 = wid * _NY + (yi >> 3) * 8
    col0 = (xi >> 7) * 128
    ry = yi & 7
    c0 = (xi & 127) & ~15
    lane = xi & 15

    pltpu.async_copy(yref.at[pl.ds(row0, 8), pl.ds(col0, 128)], tile, sem).wait()
    sel = lanes == lane
    delta = jnp.where(sel, upd, jnp.float32(0.0))
    tile[ry, pl.ds(c0, _L)] = tile[ry, pl.ds(c0, _L)] + delta
    pltpu.async_copy(tile, yref.at[pl.ds(row0, 8), pl.ds(col0, 128)], sem).wait()


_scatter_add = pl.kernel(
    _sc_body,
    out_type=(),
    mesh=plsc.VectorSubcoreMesh(core_axis_name="c", subcore_axis_name="s"),
    scratch_types=[
        pltpu.VMEM((_NSRC,), jnp.int32),
        pltpu.VMEM((_NSRC,), jnp.int32),
        pltpu.VMEM((1,), jnp.float32),
        pltpu.VMEM((1,), jnp.float32),
        pltpu.VMEM((8, 128), jnp.float32),
        pltpu.SemaphoreType.DMA,
    ],
    compiler_params=pltpu.CompilerParams(
        use_tc_tiling_on_sc=True,
        needs_layout_passes=False,
        skip_device_barrier=True,
        disable_semaphore_checks=True,
    ),
)


def kernel(Y, X, y, x, dt=1.0):
    yref = jax.new_ref(Y.reshape(_NSRC * _NY, _NX))
    _scatter_add(
        yref,
        y.astype(jnp.int32),
        x.astype(jnp.int32),
        X.astype(jnp.float32),
        jnp.asarray(dt, jnp.float32).reshape(1),
    )
    return jax.freeze(yref).reshape(_NSRC, _NY, _NX)
